# baseline (speedup 1.0000x reference)
import functools

import jax
import jax.numpy as jnp
from jax import lax
from jax.experimental import pallas as pl
from jax.experimental.pallas import tpu as pltpu
from jax.experimental.pallas import tpu_sc as plsc

N = 10000
E = 320000
D = 128
H = 128
C = 64

NCORE = 2
NSUB = 16
NW = NCORE * NSUB

K = 80
NCH = 126
NCH0 = 132
NCH1 = 120
E_PAD = NW * NCH * K
N_PAD = 10240
STRIPE = N_PAD // NSUB
DUMMY_DST = 10200


def _fill_vmem_2d(ref, rows, cols, value):
    vec = jnp.full((16,), value, dtype=jnp.float32)

    def row_body(r, carry):
        def col_body(cc, carry2):
            ref[r, pl.ds(cc * 16, 16)] = vec
            return carry2
        return lax.fori_loop(0, cols // 16, col_body, carry)

    lax.fori_loop(0, rows, row_body, 0)


DEG_W = 16


DEG_REAL_LAST = (E - (NW - 1) * NCH * K) // K


def _make_sc_degree():
    mesh = plsc.VectorSubcoreMesh(core_axis_name="c", subcore_axis_name="s")

    @functools.partial(
        pl.kernel,
        out_type=jax.ShapeDtypeStruct((NCORE, N_PAD, DEG_W), jnp.float32),
        mesh=mesh,
        compiler_params=pltpu.CompilerParams(use_tc_tiling_on_sc=False),
        scratch_types=[
            pltpu.VMEM_SHARED((N_PAD, DEG_W), jnp.float32),
            pltpu.VMEM((6, K), jnp.int32),
            pltpu.VMEM((K, DEG_W), jnp.float32),
            pltpu.VMEM((K, DEG_W), jnp.float32),
            [pltpu.SemaphoreType.DMA] * 6,
            [pltpu.SemaphoreType.DMA] * 3,
        ],
    )
    def k(edst_hbm, out_hbm, acc, islot, ones_v, zbuf, isems, ssems):
        c = lax.axis_index("c")
        s = lax.axis_index("s")
        wid = c * NSUB + s
        base = s * STRIPE
        base_e = wid * (NCH * K)
        limit = jnp.where(wid == NW - 1, DEG_REAL_LAST, NCH)
        dummy_rows = N + 200 + lax.iota(jnp.int32, 16)

        def load_idx(jj, q):
            off = jnp.minimum(base_e + jj * K, E - K)
            pltpu.async_copy(edst_hbm.at[pl.ds(off, K)], islot.at[q],
                             isems[q])

        def wait_idx(jj, q):
            off = jnp.minimum(base_e + jj * K, E - K)
            pltpu.make_async_copy(
                edst_hbm.at[pl.ds(off, K)], islot.at[q], isems[q]).wait()

            @pl.when(jj >= limit)
            def _():
                for t in range(K // 16):
                    islot[q, pl.ds(t * 16, 16)] = dummy_rows

        _fill_vmem_2d(ones_v, K, DEG_W, 1.0)
        _fill_vmem_2d(zbuf, K, DEG_W, 0.0)
        for i in range(STRIPE // K):
            pltpu.sync_copy(zbuf, acc.at[pl.ds(base + i * K, K)])
        plsc.subcore_barrier()

        for q in range(3):
            load_idx(q, q)

        def body(i, carry):
            for bb in range(6):
                j = 6 * i + bb
                sq = bb % 3
                qL = (bb + 3) % 6

                @pl.when(j >= 3)
                def _():
                    pltpu.make_async_copy(
                        ones_v, acc.at[islot.at[bb]], ssems[sq]).wait()

                wait_idx(j, bb)
                pltpu.async_copy(ones_v, acc.at[islot.at[bb]], ssems[sq],
                                 add=True)

                @pl.when(j + 3 < NCH)
                def _():
                    load_idx(j + 3, qL)
            return carry

        lax.fori_loop(0, NCH // 6, body, 0)
        for r in (3, 4, 5):
            pltpu.make_async_copy(
                ones_v, acc.at[islot.at[r]], ssems[r % 3]).wait()
        plsc.subcore_barrier()
        pltpu.sync_copy(acc.at[pl.ds(base, STRIPE)],
                        out_hbm.at[c, pl.ds(base, STRIPE)])

    return k


_sc_degree = _make_sc_degree()


REAL_LAST = (E - NSUB * NCH0 * K - (NSUB - 1) * NCH1 * K) // K


def _make_sc_scatter(width):
    mesh = plsc.VectorSubcoreMesh(core_axis_name="c", subcore_axis_name="s")

    @functools.partial(
        pl.kernel,
        out_type=jax.ShapeDtypeStruct((NCORE, N_PAD, width), jnp.float32),
        mesh=mesh,
        compiler_params=(None if width % 128 == 0 else
                         pltpu.CompilerParams(use_tc_tiling_on_sc=False)),
        scratch_types=[
            pltpu.VMEM_SHARED((N_PAD, width), jnp.float32),
            pltpu.VMEM((6, 2, K), jnp.int32),
            pltpu.VMEM((K, width), jnp.float32),
            pltpu.VMEM((K, width), jnp.float32),
            pltpu.VMEM((K, width), jnp.float32),
            [pltpu.SemaphoreType.DMA] * 6,
            [pltpu.SemaphoreType.DMA] * 3,
            [pltpu.SemaphoreType.DMA] * 3,
        ],
    )
    def k(g_hbm, esrc_hbm, edst_hbm, out_hbm,
          acc, islot, buf0, buf1, buf2, isems, gsems, ssems):
        c = lax.axis_index("c")
        s = lax.axis_index("s")
        base = s * STRIPE
        bufs = (buf0, buf1, buf2)
        nch = jnp.where(c == 0, NCH0, NCH1)
        base_e = jnp.where(c == 0, s * (NCH0 * K),
                           NSUB * NCH0 * K + s * (NCH1 * K))
        limit = jnp.where((c == 1) & (s == NSUB - 1), REAL_LAST, nch)
        dummy_rows = N + 200 + lax.iota(jnp.int32, 16)

        def load_idx(jj, q, sem):
            off = jnp.minimum(base_e + jj * K, E - K)
            pltpu.async_copy(esrc_hbm.at[pl.ds(off, K)], islot.at[q, 0], sem)
            pltpu.async_copy(edst_hbm.at[pl.ds(off, K)], islot.at[q, 1], sem)

        def wait_idx(jj, q, sem):
            off = jnp.minimum(base_e + jj * K, E - K)
            pltpu.make_async_copy(
                esrc_hbm.at[pl.ds(off, K)], islot.at[q, 0], sem).wait()
            pltpu.make_async_copy(
                edst_hbm.at[pl.ds(off, K)], islot.at[q, 1], sem).wait()

            @pl.when(jj >= limit)
            def _():
                for t in range(K // 16):
                    islot[q, 1, pl.ds(t * 16, 16)] = dummy_rows

        _fill_vmem_2d(buf0, K, width, 0.0)
        for i in range(STRIPE // K):
            pltpu.sync_copy(buf0, acc.at[pl.ds(base + i * K, K)])
        plsc.subcore_barrier()

        for q in range(3):
            load_idx(q, q, isems[q])
        for q in range(2):
            wait_idx(q, q, isems[q])
            pltpu.async_copy(g_hbm.at[islot.at[q, 0]], bufs[q], gsems[q])

        def body(i, carry):
            for bb in range(6):
                j = 6 * i + bb
                b = bb % 3
                q2 = (bb + 2) % 6
                b2 = (bb + 2) % 3
                qL = (bb + 3) % 6

                pltpu.make_async_copy(
                    g_hbm.at[islot.at[bb, 0]], bufs[b], gsems[b]).wait()
                pltpu.async_copy(
                    bufs[b], acc.at[islot.at[bb, 1]], ssems[b], add=True)

                @pl.when(j + 2 < nch)
                def _():
                    @pl.when(j >= 1)
                    def _():
                        pltpu.make_async_copy(
                            bufs[b2], acc.at[islot.at[q2, 1]],
                            ssems[b2]).wait()

                    wait_idx(j + 2, q2, isems[q2])
                    pltpu.async_copy(
                        g_hbm.at[islot.at[q2, 0]], bufs[b2], gsems[b2])

                @pl.when(j + 3 < nch)
                def _():
                    load_idx(j + 3, qL, isems[qL])
            return carry

        lax.fori_loop(0, nch // 6, body, 0)
        for r in (3, 4, 5):
            pltpu.make_async_copy(
                bufs[r % 3], acc.at[islot.at[r, 1]], ssems[r % 3]).wait()
        plsc.subcore_barrier()
        pltpu.sync_copy(acc.at[pl.ds(base, STRIPE)],
                        out_hbm.at[c, pl.ds(base, STRIPE)])

    return k


_sc_scatter_h = _make_sc_scatter(H)
_sc_scatter_c = _make_sc_scatter(C)


ROWS = 1000


def _dinv_block(degp):
    deg = degp[0, :, 0:1] + degp[1, :, 0:1] + 2.0
    return lax.rsqrt(deg)


def _gelu_exact(a):
    return 0.5 * a * (1.0 + lax.erf(a * 0.7071067811865476))


def _tc_front(x, W, degp):
    def body(x_ref, w_ref, degp_ref, h_ref, g_ref):
        dinv = _dinv_block(degp_ref)
        hb = jnp.dot(x_ref[...], w_ref[...],
                     preferred_element_type=jnp.float32)
        h_ref[...] = hb
        g_ref[...] = hb * dinv

    return pl.pallas_call(
        body,
        grid=(N // ROWS,),
        in_specs=[
            pl.BlockSpec((ROWS, D), lambda i: (i, 0)),
            pl.BlockSpec((D, H), lambda i: (0, 0)),
            pl.BlockSpec((2, ROWS, DEG_W), lambda i: (0, i, 0)),
        ],
        out_specs=[
            pl.BlockSpec((ROWS, H), lambda i: (i, 0)),
            pl.BlockSpec((ROWS, H), lambda i: (i, 0)),
        ],
        out_shape=[jax.ShapeDtypeStruct((N, H), jnp.float32)] * 2,
    )(x, W, degp)


def _tc_mid(s_part, h_prev, degp, b_prev, W_next, res, win, wout, want_z):
    have_res = res is not None

    def body(*refs):
        s_ref, h_ref, degp_ref, b_ref, w_ref = refs[:5]
        if have_res:
            r_ref = refs[5]
            outs = refs[6:]
        else:
            outs = refs[5:]
        dinv = _dinv_block(degp_ref)
        sb = s_ref[0] + s_ref[1]
        a = dinv * sb + (2.0 * dinv * dinv) * h_ref[...] + b_ref[...]
        if have_res:
            a = a + r_ref[...]
        z = _gelu_exact(a)
        hn = jnp.dot(z, w_ref[...], preferred_element_type=jnp.float32)
        gn = hn * dinv
        if want_z:
            outs[0][...] = z
            outs[1][...] = hn
            outs[2][...] = gn
        else:
            outs[0][...] = hn
            outs[1][...] = gn

    in_specs = [
        pl.BlockSpec((2, ROWS, win), lambda i: (0, i, 0)),
        pl.BlockSpec((ROWS, win), lambda i: (i, 0)),
        pl.BlockSpec((2, ROWS, DEG_W), lambda i: (0, i, 0)),
        pl.BlockSpec((1, win), lambda i: (0, 0)),
        pl.BlockSpec((win, wout), lambda i: (0, 0)),
    ]
    args = [s_part, h_prev, degp, b_prev.reshape(1, win), W_next]
    if have_res:
        in_specs.append(pl.BlockSpec((ROWS, win), lambda i: (i, 0)))
        args.append(res)

    out_shapes = ([jax.ShapeDtypeStruct((N, win), jnp.float32)] if want_z
                  else [])
    out_shapes += [jax.ShapeDtypeStruct((N, wout), jnp.float32)] * 2
    out_specs = ([pl.BlockSpec((ROWS, win), lambda i: (i, 0))] if want_z
                 else [])
    out_specs += [pl.BlockSpec((ROWS, wout), lambda i: (i, 0))] * 2

    return pl.pallas_call(
        body,
        grid=(N // ROWS,),
        in_specs=in_specs,
        out_specs=out_specs,
        out_shape=out_shapes,
    )(*args)


def _tc_final(s_part, h_prev, degp, b_prev):
    def body(s_ref, h_ref, degp_ref, b_ref, o_ref):
        dinv = _dinv_block(degp_ref)
        sb = s_ref[0] + s_ref[1]
        o_ref[...] = (dinv * sb + (2.0 * dinv * dinv) * h_ref[...]
                      + b_ref[...])

    return pl.pallas_call(
        body,
        grid=(N // ROWS,),
        in_specs=[
            pl.BlockSpec((2, ROWS, C), lambda i: (0, i, 0)),
            pl.BlockSpec((ROWS, C), lambda i: (i, 0)),
            pl.BlockSpec((2, ROWS, DEG_W), lambda i: (0, i, 0)),
            pl.BlockSpec((1, C), lambda i: (0, 0)),
        ],
        out_specs=pl.BlockSpec((ROWS, C), lambda i: (i, 0)),
        out_shape=jax.ShapeDtypeStruct((N, C), jnp.float32),
    )(s_part, h_prev, degp, b_prev.reshape(1, C))


def kernel(x, edge_index, W0, b0, W1, b1, W2, b2):
    esrc = edge_index[0]
    dst = edge_index[1]
    degp = _sc_degree(dst)

    h0, g0 = _tc_front(x, W0, degp)
    s0 = _sc_scatter_h(g0, esrc, dst)
    z0, h1, g1 = _tc_mid(s0, h0, degp, b0, W1, None, H, H, True)
    s1 = _sc_scatter_h(g1, esrc, dst)
    h2, g2 = _tc_mid(s1, h1, degp, b1, W2, z0, H, C, False)
    s2 = _sc_scatter_c(g2, esrc, dst)
    return _tc_final(s2, h2, degp, b2)


# speedup vs baseline: 1.0205x; 1.0205x over previous
import functools

import jax
import jax.numpy as jnp
from jax import lax
from jax.experimental import pallas as pl
from jax.experimental.pallas import tpu as pltpu
from jax.experimental.pallas import tpu_sc as plsc

N = 10000
E = 320000
D = 128
H = 128
C = 64

NCORE = 2
NSUB = 16
NW = NCORE * NSUB

K = 80
NCH = 126
NCH0 = 132
NCH1 = 120
E_PAD = NW * NCH * K
N_PAD = 10240
STRIPE = N_PAD // NSUB
DUMMY_DST = 10200


def _fill_vmem_2d(ref, rows, cols, value):
    vec = jnp.full((16,), value, dtype=jnp.float32)

    def row_body(r, carry):
        def col_body(cc, carry2):
            ref[r, pl.ds(cc * 16, 16)] = vec
            return carry2
        return lax.fori_loop(0, cols // 16, col_body, carry)

    lax.fori_loop(0, rows, row_body, 0)


DEG_W = 16


def _make_sc_degree():
    mesh = plsc.VectorSubcoreMesh(core_axis_name="c", subcore_axis_name="s")

    @functools.partial(
        pl.kernel,
        out_type=jax.ShapeDtypeStruct((NCORE, N_PAD, DEG_W), jnp.float32),
        mesh=mesh,
        compiler_params=pltpu.CompilerParams(use_tc_tiling_on_sc=False),
        scratch_types=[
            pltpu.VMEM_SHARED((N_PAD, DEG_W), jnp.float32),
            pltpu.VMEM((NCH, K), jnp.int32),
            pltpu.VMEM((K, DEG_W), jnp.float32),
            pltpu.VMEM((K, DEG_W), jnp.float32),
            pltpu.SemaphoreType.DMA,
            pltpu.SemaphoreType.DMA,
        ],
    )
    def k(dst_hbm, out_hbm, acc, dst_v, ones_v, zbuf, sem0, sem1):
        c = lax.axis_index("c")
        s = lax.axis_index("s")
        wid = c * NSUB + s
        base = s * STRIPE
        sems = (sem0, sem1)

        _fill_vmem_2d(ones_v, K, DEG_W, 1.0)
        _fill_vmem_2d(zbuf, K, DEG_W, 0.0)
        pltpu.sync_copy(dst_hbm.at[wid], dst_v)
        for i in range(STRIPE // K):
            pltpu.sync_copy(zbuf, acc.at[pl.ds(base + i * K, K)])
        plsc.subcore_barrier()

        def body(i, carry):
            for b in range(2):
                j = 2 * i + b

                @pl.when(i > 0)
                def _():
                    pltpu.make_async_copy(
                        ones_v, acc.at[dst_v.at[j]], sems[b]).wait()

                pltpu.async_copy(ones_v, acc.at[dst_v.at[j]], sems[b],
                                 add=True)
            return carry

        lax.fori_loop(0, NCH // 2, body, 0)
        pltpu.make_async_copy(ones_v, acc.at[dst_v.at[0]], sem0).wait()
        pltpu.make_async_copy(ones_v, acc.at[dst_v.at[1]], sem1).wait()
        plsc.subcore_barrier()
        pltpu.sync_copy(acc.at[pl.ds(base, STRIPE)],
                        out_hbm.at[c, pl.ds(base, STRIPE)])

    return k


_sc_degree = _make_sc_degree()


REAL_LAST = (E - NSUB * NCH0 * K - (NSUB - 1) * NCH1 * K) // K


def _make_sc_scatter(width):
    mesh = plsc.VectorSubcoreMesh(core_axis_name="c", subcore_axis_name="s")

    @functools.partial(
        pl.kernel,
        out_type=jax.ShapeDtypeStruct((NCORE, N_PAD, width), jnp.float32),
        mesh=mesh,
        compiler_params=(None if width % 128 == 0 else
                         pltpu.CompilerParams(use_tc_tiling_on_sc=False)),
        scratch_types=[
            pltpu.VMEM_SHARED((N_PAD, width), jnp.float32),
            pltpu.VMEM((6, 2, K), jnp.int32),
            pltpu.VMEM((K, width), jnp.float32),
            pltpu.VMEM((K, width), jnp.float32),
            pltpu.VMEM((K, width), jnp.float32),
            [pltpu.SemaphoreType.DMA] * 6,
            [pltpu.SemaphoreType.DMA] * 3,
            [pltpu.SemaphoreType.DMA] * 3,
        ],
    )
    def k(g_hbm, esrc_hbm, edst_hbm, out_hbm,
          acc, islot, buf0, buf1, buf2, isems, gsems, ssems):
        c = lax.axis_index("c")
        s = lax.axis_index("s")
        base = s * STRIPE
        bufs = (buf0, buf1, buf2)
        nch = jnp.where(c == 0, NCH0, NCH1)
        base_e = jnp.where(c == 0, s * (NCH0 * K),
                           NSUB * NCH0 * K + s * (NCH1 * K))
        limit = jnp.where((c == 1) & (s == NSUB - 1), REAL_LAST, nch)
        dummy_rows = N + 200 + lax.iota(jnp.int32, 16)

        def load_idx(jj, q, sem):
            off = jnp.minimum(base_e + jj * K, E - K)
            pltpu.async_copy(esrc_hbm.at[pl.ds(off, K)], islot.at[q, 0], sem)
            pltpu.async_copy(edst_hbm.at[pl.ds(off, K)], islot.at[q, 1], sem)

        def wait_idx(jj, q, sem):
            off = jnp.minimum(base_e + jj * K, E - K)
            pltpu.make_async_copy(
                esrc_hbm.at[pl.ds(off, K)], islot.at[q, 0], sem).wait()
            pltpu.make_async_copy(
                edst_hbm.at[pl.ds(off, K)], islot.at[q, 1], sem).wait()

            @pl.when(jj >= limit)
            def _():
                for t in range(K // 16):
                    islot[q, 1, pl.ds(t * 16, 16)] = dummy_rows

        _fill_vmem_2d(buf0, K, width, 0.0)
        for i in range(STRIPE // K):
            pltpu.sync_copy(buf0, acc.at[pl.ds(base + i * K, K)])
        plsc.subcore_barrier()

        for q in range(3):
            load_idx(q, q, isems[q])
        for q in range(2):
            wait_idx(q, q, isems[q])
            pltpu.async_copy(g_hbm.at[islot.at[q, 0]], bufs[q], gsems[q])

        def body(i, carry):
            for bb in range(6):
                j = 6 * i + bb
                b = bb % 3
                q2 = (bb + 2) % 6
                b2 = (bb + 2) % 3
                qL = (bb + 3) % 6

                pltpu.make_async_copy(
                    g_hbm.at[islot.at[bb, 0]], bufs[b], gsems[b]).wait()
                pltpu.async_copy(
                    bufs[b], acc.at[islot.at[bb, 1]], ssems[b], add=True)

                @pl.when(j + 2 < nch)
                def _():
                    @pl.when(j >= 1)
                    def _():
                        pltpu.make_async_copy(
                            bufs[b2], acc.at[islot.at[q2, 1]],
                            ssems[b2]).wait()

                    wait_idx(j + 2, q2, isems[q2])
                    pltpu.async_copy(
                        g_hbm.at[islot.at[q2, 0]], bufs[b2], gsems[b2])

                @pl.when(j + 3 < nch)
                def _():
                    load_idx(j + 3, qL, isems[qL])
            return carry

        lax.fori_loop(0, nch // 6, body, 0)
        for r in (3, 4, 5):
            pltpu.make_async_copy(
                bufs[r % 3], acc.at[islot.at[r, 1]], ssems[r % 3]).wait()
        plsc.subcore_barrier()
        pltpu.sync_copy(acc.at[pl.ds(base, STRIPE)],
                        out_hbm.at[c, pl.ds(base, STRIPE)])

    return k


_sc_scatter_h = _make_sc_scatter(H)
_sc_scatter_c = _make_sc_scatter(C)


ROWS = 1000


def _dinv_block(degp):
    deg = degp[0, :, 0:1] + degp[1, :, 0:1] + 2.0
    return lax.rsqrt(deg)


def _gelu_exact(a):
    return 0.5 * a * (1.0 + lax.erf(a * 0.7071067811865476))


def _tc_front(x, W, degp):
    def body(x_ref, w_ref, degp_ref, h_ref, g_ref):
        dinv = _dinv_block(degp_ref)
        hb = jnp.dot(x_ref[...], w_ref[...],
                     preferred_element_type=jnp.float32)
        h_ref[...] = hb
        g_ref[...] = hb * dinv

    return pl.pallas_call(
        body,
        grid=(N // ROWS,),
        in_specs=[
            pl.BlockSpec((ROWS, D), lambda i: (i, 0)),
            pl.BlockSpec((D, H), lambda i: (0, 0)),
            pl.BlockSpec((2, ROWS, DEG_W), lambda i: (0, i, 0)),
        ],
        out_specs=[
            pl.BlockSpec((ROWS, H), lambda i: (i, 0)),
            pl.BlockSpec((ROWS, H), lambda i: (i, 0)),
        ],
        out_shape=[jax.ShapeDtypeStruct((N, H), jnp.float32)] * 2,
    )(x, W, degp)


def _tc_mid(s_part, h_prev, degp, b_prev, W_next, res, win, wout, want_z):
    have_res = res is not None

    def body(*refs):
        s_ref, h_ref, degp_ref, b_ref, w_ref = refs[:5]
        if have_res:
            r_ref = refs[5]
            outs = refs[6:]
        else:
            outs = refs[5:]
        dinv = _dinv_block(degp_ref)
        sb = s_ref[0] + s_ref[1]
        a = dinv * sb + (2.0 * dinv * dinv) * h_ref[...] + b_ref[...]
        if have_res:
            a = a + r_ref[...]
        z = _gelu_exact(a)
        hn = jnp.dot(z, w_ref[...], preferred_element_type=jnp.float32)
        gn = hn * dinv
        if want_z:
            outs[0][...] = z
            outs[1][...] = hn
            outs[2][...] = gn
        else:
            outs[0][...] = hn
            outs[1][...] = gn

    in_specs = [
        pl.BlockSpec((2, ROWS, win), lambda i: (0, i, 0)),
        pl.BlockSpec((ROWS, win), lambda i: (i, 0)),
        pl.BlockSpec((2, ROWS, DEG_W), lambda i: (0, i, 0)),
        pl.BlockSpec((1, win), lambda i: (0, 0)),
        pl.BlockSpec((win, wout), lambda i: (0, 0)),
    ]
    args = [s_part, h_prev, degp, b_prev.reshape(1, win), W_next]
    if have_res:
        in_specs.append(pl.BlockSpec((ROWS, win), lambda i: (i, 0)))
        args.append(res)

    out_shapes = ([jax.ShapeDtypeStruct((N, win), jnp.float32)] if want_z
                  else [])
    out_shapes += [jax.ShapeDtypeStruct((N, wout), jnp.float32)] * 2
    out_specs = ([pl.BlockSpec((ROWS, win), lambda i: (i, 0))] if want_z
                 else [])
    out_specs += [pl.BlockSpec((ROWS, wout), lambda i: (i, 0))] * 2

    return pl.pallas_call(
        body,
        grid=(N // ROWS,),
        in_specs=in_specs,
        out_specs=out_specs,
        out_shape=out_shapes,
    )(*args)


def _tc_final(s_part, h_prev, degp, b_prev):
    def body(s_ref, h_ref, degp_ref, b_ref, o_ref):
        dinv = _dinv_block(degp_ref)
        sb = s_ref[0] + s_ref[1]
        o_ref[...] = (dinv * sb + (2.0 * dinv * dinv) * h_ref[...]
                      + b_ref[...])

    return pl.pallas_call(
        body,
        grid=(N // ROWS,),
        in_specs=[
            pl.BlockSpec((2, ROWS, C), lambda i: (0, i, 0)),
            pl.BlockSpec((ROWS, C), lambda i: (i, 0)),
            pl.BlockSpec((2, ROWS, DEG_W), lambda i: (0, i, 0)),
            pl.BlockSpec((1, C), lambda i: (0, 0)),
        ],
        out_specs=pl.BlockSpec((ROWS, C), lambda i: (i, 0)),
        out_shape=jax.ShapeDtypeStruct((N, C), jnp.float32),
    )(s_part, h_prev, degp, b_prev.reshape(1, C))


def kernel(x, edge_index, W0, b0, W1, b1, W2, b2):
    esrc = edge_index[0]
    dst = edge_index[1]
    pad = E_PAD - E
    dummy = N + (jnp.arange(pad, dtype=jnp.int32) % (N_PAD - N))
    dst_p = jnp.concatenate([dst, dummy]).reshape(NW, NCH, K)
    degp = _sc_degree(dst_p)

    h0, g0 = _tc_front(x, W0, degp)
    s0 = _sc_scatter_h(g0, esrc, dst)
    z0, h1, g1 = _tc_mid(s0, h0, degp, b0, W1, None, H, H, True)
    s1 = _sc_scatter_h(g1, esrc, dst)
    h2, g2 = _tc_mid(s1, h1, degp, b1, W2, z0, H, C, False)
    s2 = _sc_scatter_c(g2, esrc, dst)
    return _tc_final(s2, h2, degp, b2)
